# baseline (speedup 1.0000x reference)
import functools

import jax
import jax.numpy as jnp
from jax import lax
from jax.experimental import pallas as pl
from jax.experimental.pallas import tpu as pltpu
from jax.experimental.pallas import tpu_sc as plsc

NN = 10000
NE = 320000
D = 128
ST = 4
NLAY = 2
NG = 16

NW = 32
EPW = NE // NW
CH = 80
NCH = EPW // CH
RPW = 624
CHS = 64
RCAP = 10240

_mesh = plsc.VectorSubcoreMesh(core_axis_name="c", subcore_axis_name="s")


@functools.partial(
    pl.kernel,
    mesh=_mesh,
    out_type=[
        jax.ShapeDtypeStruct((NN, D), jnp.float32),
        jax.ShapeDtypeStruct((NW * ST * NN,), jnp.float32),
        jax.ShapeDtypeStruct((NW * RCAP,), jnp.int32),
        jax.ShapeDtypeStruct((NW * RCAP,), jnp.int32),
        jax.ShapeDtypeStruct((NW * RCAP,), jnp.float32),
        jax.ShapeDtypeStruct((NW * 16,), jnp.int32),
    ],
    scratch_types=[
        pltpu.VMEM((CH,), jnp.int32),
        pltpu.VMEM((CH, D), jnp.float32),
        pltpu.VMEM((EPW,), jnp.int32),
        pltpu.VMEM((EPW,), jnp.int32),
        pltpu.VMEM((EPW,), jnp.int32),
        pltpu.VMEM((EPW,), jnp.float32),
        pltpu.VMEM((RCAP,), jnp.int32),
        pltpu.VMEM((RCAP,), jnp.int32),
        pltpu.VMEM((RCAP,), jnp.float32),
        pltpu.VMEM((ST * NN,), jnp.float32),
        pltpu.VMEM((16,), jnp.int32),
        pltpu.SemaphoreType.DMA,
    ],
    compiler_params=pltpu.CompilerParams(needs_layout_passes=False),
)
def _sc_embed_touched(emb_hbm, ids_hbm, src_hbm, dst_hbm, time_hbm, w_hbm,
                      x_out, touched_out, ps_out, pd_out, pw_out, oc_out,
                      idx_v, rows_v, es_v, ed_v, et_v, ew_v,
                      ps_v, pd_v, pw_v, tfl_v, oc_v, sem):
    c = lax.axis_index("c")
    s = lax.axis_index("s")
    wid = c * 16 + s

    nchunk_all = NN // CH
    z16 = jnp.zeros((16,), jnp.float32)

    def emb_body(i, _):
        c2 = wid + i * NW
        base = c2 * CH
        pltpu.sync_copy(ids_hbm.at[pl.ds(base, CH)], idx_v)
        pltpu.async_copy(emb_hbm.at[idx_v], rows_v, sem).wait()
        pltpu.sync_copy(rows_v, x_out.at[pl.ds(base, CH)])
        return 0

    my_chunks = (nchunk_all - wid + NW - 1) // NW
    lax.fori_loop(0, my_chunks, emb_body, 0)

    ebase = wid * EPW
    pltpu.sync_copy(src_hbm.at[pl.ds(ebase, EPW)], es_v)
    pltpu.sync_copy(dst_hbm.at[pl.ds(ebase, EPW)], ed_v)
    pltpu.sync_copy(time_hbm.at[pl.ds(ebase, EPW)], et_v)
    pltpu.sync_copy(w_hbm.at[pl.ds(ebase, EPW)], ew_v)

    def zero_body(k, _):
        tfl_v[pl.ds(k * 16, 16)] = z16
        return 0
    lax.fori_loop(0, ST * NN // 16, zero_body, 0)

    ones = jnp.ones((16,), jnp.float32)

    def edge_body(v, _):
        sl = pl.ds(v * 16, 16)
        tbase = et_v[sl] * NN
        plsc.addupdate_scatter(tfl_v, [tbase + es_v[sl]], ones)
        plsc.addupdate_scatter(tfl_v, [tbase + ed_v[sl]], ones)
        return 0

    lax.fori_loop(0, EPW // 16, edge_body, 0)
    pltpu.sync_copy(tfl_v, touched_out.at[pl.ds(wid * (ST * NN), ST * NN)])

    iota16 = jnp.arange(16, dtype=jnp.int32)

    def pre_body(k, _):
        sl = pl.ds(k * 16, 16)
        spread = (iota16 * 599 + k * 23 + wid * 311) % jnp.int32(NN)
        ps_v[sl] = spread
        pd_v[sl] = spread
        pw_v[sl] = z16
        return 0
    lax.fori_loop(0, RCAP // 16, pre_body, 0)
    offcnt = []
    pos = jnp.int32(0)
    for t in range(ST):
        pos = (pos + CHS - 1) & ~jnp.int32(CHS - 1)
        seg_start = pos

        def comp_body(v, p, t=t):
            sl = pl.ds(v * 16, 16)
            tv = et_v[sl]
            m = tv == t
            csum = jnp.cumsum(m.astype(jnp.int32))
            dpos = p + csum - 1
            plsc.store_scatter(ps_v, [dpos], es_v[sl], mask=m)
            plsc.store_scatter(pd_v, [dpos], ed_v[sl], mask=m)
            plsc.store_scatter(pw_v, [dpos], ew_v[sl], mask=m)
            return p + csum[15]

        pos = lax.fori_loop(0, EPW // 16, comp_body, pos)
        offcnt.append(seg_start)
        offcnt.append(pos - seg_start)

    pltpu.sync_copy(ps_v, ps_out.at[pl.ds(wid * RCAP, RCAP)])
    pltpu.sync_copy(pd_v, pd_out.at[pl.ds(wid * RCAP, RCAP)])
    pltpu.sync_copy(pw_v, pw_out.at[pl.ds(wid * RCAP, RCAP)])
    ocv = jnp.zeros((16,), jnp.int32)
    for k, val in enumerate(offcnt):
        ocv = jnp.where(iota16 == k, val, ocv)
    oc_v[...] = ocv
    pltpu.sync_copy(oc_v, oc_out.at[pl.ds(wid * 16, 16)])


def _make_sc_scatter(t):
    @functools.partial(
        pl.kernel,
        mesh=_mesh,
        out_type=jax.ShapeDtypeStruct((2, NN, D), jnp.float32),
        scratch_types=[
            pltpu.VMEM((RCAP,), jnp.int32),
            pltpu.VMEM((RCAP,), jnp.int32),
            pltpu.VMEM((RCAP,), jnp.float32),
            pltpu.VMEM((CHS,), jnp.int32),
            pltpu.VMEM((CHS,), jnp.int32),
            pltpu.VMEM((CHS, D), jnp.float32),
            pltpu.VMEM((CHS,), jnp.int32),
            pltpu.VMEM((CHS,), jnp.int32),
            pltpu.VMEM((CHS, D), jnp.float32),
            pltpu.VMEM((16,), jnp.int32),
            pltpu.VMEM_SHARED((NN, D), jnp.float32),
            pltpu.SemaphoreType.DMA,
            pltpu.SemaphoreType.DMA,
            pltpu.SemaphoreType.DMA,
            pltpu.SemaphoreType.DMA,
        ],
    )
    def _sc_scatter(ht_hbm, ps_hbm, pd_hbm, pw_hbm, oc_hbm, zeros_hbm,
                    agg_out, rs_v, rd_v, rw_v,
                    src_a, dst_a, rows_a, src_b, dst_b, rows_b,
                    oc_v, agg_sh, sem_a, sem_b, sem_sa, sem_sb):
        c = lax.axis_index("c")
        s = lax.axis_index("s")
        wid = c * 16 + s

        pltpu.sync_copy(zeros_hbm.at[pl.ds(s * RPW, RPW)],
                        agg_sh.at[pl.ds(s * RPW, RPW)])

        @pl.when(s == 15)
        def _():
            pltpu.sync_copy(zeros_hbm.at[pl.ds(16 * RPW, NN - 16 * RPW)],
                            agg_sh.at[pl.ds(16 * RPW, NN - 16 * RPW)])

        pltpu.sync_copy(oc_hbm.at[pl.ds(wid * 16, 16)], oc_v)
        pltpu.async_copy(ps_hbm.at[pl.ds(wid * RCAP, RCAP)], rs_v, sem_a)
        pltpu.async_copy(pd_hbm.at[pl.ds(wid * RCAP, RCAP)], rd_v, sem_a)
        pltpu.async_copy(pw_hbm.at[pl.ds(wid * RCAP, RCAP)], rw_v, sem_a)
        pltpu.make_async_copy(ps_hbm.at[pl.ds(wid * RCAP, RCAP)], rs_v,
                              sem_a).wait()
        pltpu.make_async_copy(pd_hbm.at[pl.ds(wid * RCAP, RCAP)], rd_v,
                              sem_a).wait()
        pltpu.make_async_copy(pw_hbm.at[pl.ds(wid * RCAP, RCAP)], rw_v,
                              sem_a).wait()
        ocv = oc_v[...]
        off = pl.multiple_of(ocv[2 * t], CHS)
        cnt = ocv[2 * t + 1]
        plsc.subcore_barrier()

        nchunks = (cnt + CHS - 1) // CHS

        def fire(src_v, rows_v, sem):
            pltpu.async_copy(ht_hbm.at[src_v], rows_v, sem)

        def prep(c2, src_v, dst_v):
            cbase = off + c2 * CHS
            for v in range(CHS // 16):
                osl = pl.ds(v * 16, 16)
                src_v[osl] = rs_v[pl.ds(cbase + v * 16, 16)]
                dst_v[osl] = rd_v[pl.ds(cbase + v * 16, 16)]

        def drain(src_v, rows_v, sem):
            pltpu.make_async_copy(ht_hbm.at[src_v], rows_v, sem).wait()

        def scale(c2, rows_v):
            cbase = off + c2 * CHS

            def scale_body(g, _):
                wv = rw_v[pl.ds(cbase + g * 16, 16)]
                for l in range(16):
                    e = g * 16 + l
                    we = wv[l]
                    for j in range(D // 16):
                        sl = pl.ds(j * 16, 16)
                        rows_v[e, sl] = rows_v[e, sl] * we
                return 0
            lax.fori_loop(0, CHS // 16, scale_body, 0)

        def fire_sc(dst_v, rows_v, sem):
            pltpu.async_copy(rows_v, agg_sh.at[dst_v], sem, add=True)

        def drain_sc(dst_v, rows_v, sem):
            pltpu.make_async_copy(rows_v, agg_sh.at[dst_v], sem).wait()

        @pl.when(nchunks > 0)
        def _():
            prep(jnp.int32(0), src_a, dst_a)
            fire(src_a, rows_a, sem_a)

        def pair_body(i, _):
            a = 2 * i
            b = a + 1

            @pl.when(b < nchunks)
            def _():
                @pl.when(i > 0)
                def _():
                    drain_sc(dst_b, rows_b, sem_sb)
                prep(b, src_b, dst_b)
                fire(src_b, rows_b, sem_b)
            drain(src_a, rows_a, sem_a)
            scale(a, rows_a)
            fire_sc(dst_a, rows_a, sem_sa)

            @pl.when(b < nchunks)
            def _():
                drain(src_b, rows_b, sem_b)
                scale(b, rows_b)
                fire_sc(dst_b, rows_b, sem_sb)

            @pl.when(a + 2 < nchunks)
            def _():
                drain_sc(dst_a, rows_a, sem_sa)
                prep(a + 2, src_a, dst_a)
                fire(src_a, rows_a, sem_a)
            return 0

        lax.fori_loop(0, (nchunks + 1) // 2, pair_body, 0)

        @pl.when(nchunks > 0)
        def _():
            drain_sc(dst_a, rows_a, sem_sa)

        @pl.when(nchunks > 1)
        def _():
            drain_sc(dst_b, rows_b, sem_sb)
        plsc.subcore_barrier()

        pltpu.sync_copy(agg_sh.at[pl.ds(s * RPW, RPW)],
                        agg_out.at[c, pl.ds(s * RPW, RPW)])

        @pl.when(s == 15)
        def _():
            pltpu.sync_copy(agg_sh.at[pl.ds(16 * RPW, NN - 16 * RPW)],
                            agg_out.at[c, pl.ds(16 * RPW, NN - 16 * RPW)])

    return _sc_scatter


_sc_scatter_t = [_make_sc_scatter(t) for t in range(ST)]


def _lstm_body(w_ref, wih_ref, whh_ref, bih_ref, bhh_ref, hx_ref):
    dn_t = (((1,), (1,)), ((), ()))

    def cell(x, h, c_st, wih, whh, bih, bhh):
        gates = lax.dot_general(x, wih, dn_t,
                                preferred_element_type=jnp.float32) + bih
        if h is not None:
            gates = gates + lax.dot_general(h, whh, dn_t,
                                            preferred_element_type=jnp.float32)
        gates = gates + bhh
        i_g = gates[:, 0 * D:1 * D]
        f_g = gates[:, 1 * D:2 * D]
        g_g = gates[:, 2 * D:3 * D]
        o_g = gates[:, 3 * D:4 * D]
        c2 = jax.nn.sigmoid(i_g) * jnp.tanh(g_g)
        if c_st is not None:
            c2 = c2 + jax.nn.sigmoid(f_g) * c_st
        h2 = jax.nn.sigmoid(o_g) * jnp.tanh(c2)
        return h2, c2

    for i in range(NLAY):
        wih = wih_ref[i]
        whh = whh_ref[i]
        bih = bih_ref[i]
        bhh = bhh_ref[i]
        hx = w_ref[i]
        cx_first = None
        for t in range(ST):
            if t == 0:
                hx, cx_first = cell(hx, None, None, wih, whh, bih, bhh)
            else:
                hx, _ = cell(hx, hx, cx_first, wih, whh, bih, bhh)
            hx_ref[t, i] = hx


def _tc_lstm(weights, lstm_W_ih, lstm_W_hh, lstm_b_ih, lstm_b_hh):
    return pl.pallas_call(
        _lstm_body,
        out_shape=jax.ShapeDtypeStruct((ST, NLAY, D, D), jnp.float32),
    )(weights, lstm_W_ih, lstm_W_hh,
      lstm_b_ih.reshape(NLAY, 1, 4 * D), lstm_b_hh.reshape(NLAY, 1, 4 * D))


_RB = 1000


def _adapt_body(x_ref, aw_ref, ab_ref, hx_ref, h_ref, ht_ref):
    h = lax.dot_general(x_ref[...], aw_ref[...], (((1,), (1,)), ((), ())),
                        preferred_element_type=jnp.float32) + ab_ref[...]
    h_ref[...] = h
    ht_ref[...] = jnp.dot(h, hx_ref[...], preferred_element_type=jnp.float32)


def _tc_adapt(x_emb, adapt_W, adapt_b, hx00):
    grid = NN // _RB
    return pl.pallas_call(
        _adapt_body,
        grid=(grid,),
        in_specs=[
            pl.BlockSpec((_RB, D), lambda i: (i, 0)),
            pl.BlockSpec((D, D), lambda i: (0, 0)),
            pl.BlockSpec((1, D), lambda i: (0, 0)),
            pl.BlockSpec((D, D), lambda i: (0, 0)),
        ],
        out_specs=[
            pl.BlockSpec((_RB, D), lambda i: (i, 0)),
            pl.BlockSpec((_RB, D), lambda i: (i, 0)),
        ],
        out_shape=[
            jax.ShapeDtypeStruct((NN, D), jnp.float32),
            jax.ShapeDtypeStruct((NN, D), jnp.float32),
        ],
    )(x_emb, adapt_W, adapt_b.reshape(1, D), hx00)


def _mid_body(agg_ref, hx_ref, ht_ref):
    h1 = jnp.maximum(agg_ref[0] + agg_ref[1], 0.0)
    ht_ref[...] = jnp.dot(h1, hx_ref[...], preferred_element_type=jnp.float32)


def _tc_mid(aggp, hx_next):
    grid = NN // _RB
    return pl.pallas_call(
        _mid_body,
        grid=(grid,),
        in_specs=[
            pl.BlockSpec((2, _RB, D), lambda i: (0, i, 0)),
            pl.BlockSpec((D, D), lambda i: (0, 0)),
        ],
        out_specs=pl.BlockSpec((_RB, D), lambda i: (i, 0)),
        out_shape=jax.ShapeDtypeStruct((NN, D), jnp.float32),
    )(aggp, hx_next)


def _bnd_body(agg_ref, tch_ref, hprev_ref, hx_ref, h_ref, ht_ref):
    h2 = jnp.maximum(agg_ref[0] + agg_ref[1], 0.0)
    tch = jnp.sum(tch_ref[...], axis=0) > 0.0
    h_new = jnp.where(tch, h2, hprev_ref[...])
    h_ref[...] = h_new
    ht_ref[...] = jnp.dot(h_new, hx_ref[...],
                          preferred_element_type=jnp.float32)


def _tc_boundary(aggp, tch, h_prev, hx_next):
    grid = NN // _RB
    return pl.pallas_call(
        _bnd_body,
        grid=(grid,),
        in_specs=[
            pl.BlockSpec((2, _RB, D), lambda i: (0, i, 0)),
            pl.BlockSpec((NW, _RB, 1), lambda i: (0, i, 0)),
            pl.BlockSpec((_RB, D), lambda i: (i, 0)),
            pl.BlockSpec((D, D), lambda i: (0, 0)),
        ],
        out_specs=[
            pl.BlockSpec((_RB, D), lambda i: (i, 0)),
            pl.BlockSpec((_RB, D), lambda i: (i, 0)),
        ],
        out_shape=[
            jax.ShapeDtypeStruct((NN, D), jnp.float32),
            jax.ShapeDtypeStruct((NN, D), jnp.float32),
        ],
    )(aggp, tch, h_prev, hx_next)


def _final_body(agg_ref, tch_ref, hprev_ref, gid_ref, y_ref, ow_ref, ob_ref,
                loss_ref, probs_ref, acc_ref):
    i = pl.program_id(0)

    @pl.when(i == 0)
    def _():
        acc_ref[...] = jnp.full((NG, D), -jnp.inf, jnp.float32)

    h2 = jnp.maximum(agg_ref[0] + agg_ref[1], 0.0)
    tch = jnp.sum(tch_ref[...], axis=0) > 0.0
    h_new = jnp.where(tch, h2, hprev_ref[...])
    gid = gid_ref[...]
    for g in range(NG):
        m = jnp.where(gid == g, h_new, -jnp.inf)
        acc_ref[g:g + 1, :] = jnp.maximum(
            acc_ref[g:g + 1, :], jnp.max(m, axis=0, keepdims=True))

    @pl.when(i == pl.num_programs(0) - 1)
    def _():
        pooled = acc_ref[...]
        logits = (jnp.sum(pooled * ow_ref[...], axis=1, keepdims=True)
                  + ob_ref[0, 0])
        x = logits
        y = y_ref[...]
        per = (jnp.maximum(x, 0.0) - x * y
               + jnp.log(1.0 + jnp.exp(-jnp.abs(x))))
        loss_ref[...] = jnp.sum(per, axis=0, keepdims=True) / NG
        probs_ref[...] = jax.nn.sigmoid(logits)


def _tc_final(aggp, tch, h_prev, gid, y, out_W, out_b):
    grid = NN // _RB
    return pl.pallas_call(
        _final_body,
        grid=(grid,),
        in_specs=[
            pl.BlockSpec((2, _RB, D), lambda i: (0, i, 0)),
            pl.BlockSpec((NW, _RB, 1), lambda i: (0, i, 0)),
            pl.BlockSpec((_RB, D), lambda i: (i, 0)),
            pl.BlockSpec((_RB, 1), lambda i: (i, 0)),
            pl.BlockSpec((NG, 1), lambda i: (0, 0)),
            pl.BlockSpec((1, D), lambda i: (0, 0)),
            pl.BlockSpec((1, 1), lambda i: (0, 0)),
        ],
        out_specs=[
            pl.BlockSpec((1, 1), lambda i: (0, 0)),
            pl.BlockSpec((NG, 1), lambda i: (0, 0)),
        ],
        out_shape=[
            jax.ShapeDtypeStruct((1, 1), jnp.float32),
            jax.ShapeDtypeStruct((NG, 1), jnp.float32),
        ],
        scratch_shapes=[pltpu.VMEM((NG, D), jnp.float32)],
    )(aggp, tch, h_prev, gid, y, out_W, out_b)


def kernel(node_ids, edge_src, edge_dst, edge_time, edge_weight, graph_ids,
           y_data, word_embeds, adapt_W, adapt_b, weights, lstm_W_ih,
           lstm_W_hh, lstm_b_ih, lstm_b_hh, out_W, out_b):
    node_ids = node_ids.astype(jnp.int32)
    edge_src = edge_src.astype(jnp.int32)
    edge_dst = edge_dst.astype(jnp.int32)
    edge_time = edge_time.astype(jnp.int32)
    gid = graph_ids.astype(jnp.int32).reshape(NN, 1)

    x_emb, touched_p, psrc, pdst, pwgt, offcnt = _sc_embed_touched(
        word_embeds, node_ids, edge_src, edge_dst, edge_time, edge_weight)

    hx_all = _tc_lstm(weights, lstm_W_ih, lstm_W_hh, lstm_b_ih, lstm_b_hh)
    hx = [[hx_all[t, i] for i in range(NLAY)] for t in range(ST)]

    h, ht = _tc_adapt(x_emb, adapt_W, adapt_b, hx[0][0])

    zeros = jnp.zeros((NN, D), jnp.float32)
    touched_r = touched_p.reshape(NW, ST, NN, 1)

    loss = probs = None
    for t in range(ST):
        for i in range(NLAY):
            aggp = _sc_scatter_t[t](ht, psrc, pdst, pwgt, offcnt, zeros)
            if i + 1 < NLAY:
                ht = _tc_mid(aggp, hx[t][i + 1])
            elif t + 1 < ST:
                h, ht = _tc_boundary(aggp, touched_r[:, t], h, hx[t + 1][0])
            else:
                loss, probs = _tc_final(
                    aggp, touched_r[:, t], h, gid,
                    y_data.reshape(NG, 1), out_W, out_b.reshape(1, 1))

    return loss.reshape(()), probs


# speedup vs baseline: 1.0139x; 1.0139x over previous
import functools

import jax
import jax.numpy as jnp
from jax import lax
from jax.experimental import pallas as pl
from jax.experimental.pallas import tpu as pltpu
from jax.experimental.pallas import tpu_sc as plsc

NN = 10000
NE = 320000
D = 128
ST = 4
NLAY = 2
NG = 16

NW = 32
EPW = NE // NW
CH = 80
NCH = EPW // CH
RPW = 624
CHS = 64
RCAP = 10240

_mesh = plsc.VectorSubcoreMesh(core_axis_name="c", subcore_axis_name="s")


@functools.partial(
    pl.kernel,
    mesh=_mesh,
    out_type=[
        jax.ShapeDtypeStruct((NN, D), jnp.float32),
        jax.ShapeDtypeStruct((NW * ST * NN,), jnp.float32),
        jax.ShapeDtypeStruct((NW * RCAP,), jnp.int32),
        jax.ShapeDtypeStruct((NW * RCAP,), jnp.int32),
        jax.ShapeDtypeStruct((NW * RCAP,), jnp.float32),
        jax.ShapeDtypeStruct((NW * 16,), jnp.int32),
    ],
    scratch_types=[
        pltpu.VMEM((CH,), jnp.int32),
        pltpu.VMEM((CH, D), jnp.float32),
        pltpu.VMEM((EPW,), jnp.int32),
        pltpu.VMEM((EPW,), jnp.int32),
        pltpu.VMEM((EPW,), jnp.int32),
        pltpu.VMEM((EPW,), jnp.float32),
        pltpu.VMEM((RCAP,), jnp.int32),
        pltpu.VMEM((RCAP,), jnp.int32),
        pltpu.VMEM((RCAP,), jnp.float32),
        pltpu.VMEM((ST * NN,), jnp.float32),
        pltpu.VMEM((16,), jnp.int32),
        pltpu.SemaphoreType.DMA,
    ],
    compiler_params=pltpu.CompilerParams(needs_layout_passes=False),
)
def _sc_embed_touched(emb_hbm, ids_hbm, src_hbm, dst_hbm, time_hbm, w_hbm,
                      x_out, touched_out, ps_out, pd_out, pw_out, oc_out,
                      idx_v, rows_v, es_v, ed_v, et_v, ew_v,
                      ps_v, pd_v, pw_v, tfl_v, oc_v, sem):
    c = lax.axis_index("c")
    s = lax.axis_index("s")
    wid = c * 16 + s

    nchunk_all = NN // CH
    z16 = jnp.zeros((16,), jnp.float32)

    def emb_body(i, _):
        c2 = wid + i * NW
        base = c2 * CH
        pltpu.sync_copy(ids_hbm.at[pl.ds(base, CH)], idx_v)
        pltpu.async_copy(emb_hbm.at[idx_v], rows_v, sem).wait()
        pltpu.sync_copy(rows_v, x_out.at[pl.ds(base, CH)])
        return 0

    my_chunks = (nchunk_all - wid + NW - 1) // NW
    lax.fori_loop(0, my_chunks, emb_body, 0)

    ebase = wid * EPW
    pltpu.sync_copy(src_hbm.at[pl.ds(ebase, EPW)], es_v)
    pltpu.sync_copy(dst_hbm.at[pl.ds(ebase, EPW)], ed_v)
    pltpu.sync_copy(time_hbm.at[pl.ds(ebase, EPW)], et_v)
    pltpu.sync_copy(w_hbm.at[pl.ds(ebase, EPW)], ew_v)

    def zero_body(k, _):
        tfl_v[pl.ds(k * 16, 16)] = z16
        return 0
    lax.fori_loop(0, ST * NN // 16, zero_body, 0)

    ones = jnp.ones((16,), jnp.float32)

    def edge_body(v, _):
        sl = pl.ds(v * 16, 16)
        tbase = et_v[sl] * NN
        plsc.addupdate_scatter(tfl_v, [tbase + es_v[sl]], ones)
        plsc.addupdate_scatter(tfl_v, [tbase + ed_v[sl]], ones)
        return 0

    lax.fori_loop(0, EPW // 16, edge_body, 0)
    pltpu.sync_copy(tfl_v, touched_out.at[pl.ds(wid * (ST * NN), ST * NN)])

    iota16 = jnp.arange(16, dtype=jnp.int32)

    def pre_body(k, _):
        sl = pl.ds(k * 16, 16)
        spread = (iota16 * 599 + k * 23 + wid * 311) % jnp.int32(NN)
        ps_v[sl] = spread
        pd_v[sl] = spread
        pw_v[sl] = z16
        return 0
    lax.fori_loop(0, RCAP // 16, pre_body, 0)
    offcnt = []
    pos = jnp.int32(0)
    for t in range(ST):
        pos = ((pos + CHS - 1) // CHS) * CHS
        seg_start = pos

        def comp_body(v, p, t=t):
            sl = pl.ds(v * 16, 16)
            tv = et_v[sl]
            m = tv == t
            csum = jnp.cumsum(m.astype(jnp.int32))
            dpos = p + csum - 1
            plsc.store_scatter(ps_v, [dpos], es_v[sl], mask=m)
            plsc.store_scatter(pd_v, [dpos], ed_v[sl], mask=m)
            plsc.store_scatter(pw_v, [dpos], ew_v[sl], mask=m)
            return p + csum[15]

        pos = lax.fori_loop(0, EPW // 16, comp_body, pos)
        offcnt.append(seg_start)
        offcnt.append(pos - seg_start)

    pltpu.sync_copy(ps_v, ps_out.at[pl.ds(wid * RCAP, RCAP)])
    pltpu.sync_copy(pd_v, pd_out.at[pl.ds(wid * RCAP, RCAP)])
    pltpu.sync_copy(pw_v, pw_out.at[pl.ds(wid * RCAP, RCAP)])
    ocv = jnp.zeros((16,), jnp.int32)
    for k, val in enumerate(offcnt):
        ocv = jnp.where(iota16 == k, val, ocv)
    oc_v[...] = ocv
    pltpu.sync_copy(oc_v, oc_out.at[pl.ds(wid * 16, 16)])


def _make_sc_scatter(t):
    @functools.partial(
        pl.kernel,
        mesh=_mesh,
        out_type=jax.ShapeDtypeStruct((2, NN, D), jnp.float32),
        scratch_types=[
            pltpu.VMEM((RCAP,), jnp.int32),
            pltpu.VMEM((RCAP,), jnp.int32),
            pltpu.VMEM((RCAP,), jnp.float32),
            pltpu.VMEM((CHS,), jnp.int32),
            pltpu.VMEM((CHS,), jnp.int32),
            pltpu.VMEM((CHS, D), jnp.float32),
            pltpu.VMEM((CHS,), jnp.int32),
            pltpu.VMEM((CHS,), jnp.int32),
            pltpu.VMEM((CHS, D), jnp.float32),
            pltpu.VMEM((16,), jnp.int32),
            pltpu.VMEM_SHARED((NN, D), jnp.float32),
            pltpu.SemaphoreType.DMA,
            pltpu.SemaphoreType.DMA,
            pltpu.SemaphoreType.DMA,
            pltpu.SemaphoreType.DMA,
        ],
    )
    def _sc_scatter(ht_hbm, ps_hbm, pd_hbm, pw_hbm, oc_hbm, zeros_hbm,
                    agg_out, rs_v, rd_v, rw_v,
                    src_a, dst_a, rows_a, src_b, dst_b, rows_b,
                    oc_v, agg_sh, sem_a, sem_b, sem_sa, sem_sb):
        c = lax.axis_index("c")
        s = lax.axis_index("s")
        wid = c * 16 + s

        pltpu.sync_copy(zeros_hbm.at[pl.ds(s * RPW, RPW)],
                        agg_sh.at[pl.ds(s * RPW, RPW)])

        @pl.when(s == 15)
        def _():
            pltpu.sync_copy(zeros_hbm.at[pl.ds(16 * RPW, NN - 16 * RPW)],
                            agg_sh.at[pl.ds(16 * RPW, NN - 16 * RPW)])

        pltpu.sync_copy(oc_hbm.at[pl.ds(wid * 16, 16)], oc_v)
        pltpu.async_copy(ps_hbm.at[pl.ds(wid * RCAP, RCAP)], rs_v, sem_a)
        pltpu.async_copy(pd_hbm.at[pl.ds(wid * RCAP, RCAP)], rd_v, sem_a)
        pltpu.async_copy(pw_hbm.at[pl.ds(wid * RCAP, RCAP)], rw_v, sem_a)
        pltpu.make_async_copy(ps_hbm.at[pl.ds(wid * RCAP, RCAP)], rs_v,
                              sem_a).wait()
        pltpu.make_async_copy(pd_hbm.at[pl.ds(wid * RCAP, RCAP)], rd_v,
                              sem_a).wait()
        pltpu.make_async_copy(pw_hbm.at[pl.ds(wid * RCAP, RCAP)], rw_v,
                              sem_a).wait()
        ocv = oc_v[...]
        off = pl.multiple_of(ocv[2 * t], CHS)
        cnt = ocv[2 * t + 1]
        plsc.subcore_barrier()

        nchunks = (cnt + CHS - 1) // CHS

        def fire(src_v, rows_v, sem):
            pltpu.async_copy(ht_hbm.at[src_v], rows_v, sem)

        def prep(c2, src_v, dst_v):
            cbase = off + c2 * CHS
            for v in range(CHS // 16):
                osl = pl.ds(v * 16, 16)
                src_v[osl] = rs_v[pl.ds(cbase + v * 16, 16)]
                dst_v[osl] = rd_v[pl.ds(cbase + v * 16, 16)]

        def drain(src_v, rows_v, sem):
            pltpu.make_async_copy(ht_hbm.at[src_v], rows_v, sem).wait()

        def scale(c2, rows_v):
            cbase = off + c2 * CHS

            def scale_body(g, _):
                wv = rw_v[pl.ds(cbase + g * 16, 16)]
                for l in range(16):
                    e = g * 16 + l
                    we = wv[l]
                    for j in range(D // 16):
                        sl = pl.ds(j * 16, 16)
                        rows_v[e, sl] = rows_v[e, sl] * we
                return 0
            lax.fori_loop(0, CHS // 16, scale_body, 0)

        def consume(c2, dst_v, rows_v):
            scale(c2, rows_v)
            pltpu.sync_copy(rows_v, agg_sh.at[dst_v], add=True)

        @pl.when(nchunks > 0)
        def _():
            prep(jnp.int32(0), src_a, dst_a)
            fire(src_a, rows_a, sem_a)

        def pair_body(i, _):
            a = 2 * i
            b = a + 1

            @pl.when(b < nchunks)
            def _():
                prep(b, src_b, dst_b)
                fire(src_b, rows_b, sem_b)
            drain(src_a, rows_a, sem_a)
            consume(a, dst_a, rows_a)

            @pl.when(a + 2 < nchunks)
            def _():
                prep(a + 2, src_a, dst_a)
                fire(src_a, rows_a, sem_a)

            @pl.when(b < nchunks)
            def _():
                drain(src_b, rows_b, sem_b)
                consume(b, dst_b, rows_b)
            return 0

        lax.fori_loop(0, (nchunks + 1) // 2, pair_body, 0)
        plsc.subcore_barrier()

        pltpu.sync_copy(agg_sh.at[pl.ds(s * RPW, RPW)],
                        agg_out.at[c, pl.ds(s * RPW, RPW)])

        @pl.when(s == 15)
        def _():
            pltpu.sync_copy(agg_sh.at[pl.ds(16 * RPW, NN - 16 * RPW)],
                            agg_out.at[c, pl.ds(16 * RPW, NN - 16 * RPW)])

    return _sc_scatter


_sc_scatter_t = [_make_sc_scatter(t) for t in range(ST)]


def _lstm_body(w_ref, wih_ref, whh_ref, bih_ref, bhh_ref, hx_ref):
    dn_t = (((1,), (1,)), ((), ()))

    def cell(x, h, c_st, wih, whh, bih, bhh):
        gates = lax.dot_general(x, wih, dn_t,
                                preferred_element_type=jnp.float32) + bih
        if h is not None:
            gates = gates + lax.dot_general(h, whh, dn_t,
                                            preferred_element_type=jnp.float32)
        gates = gates + bhh
        i_g = gates[:, 0 * D:1 * D]
        f_g = gates[:, 1 * D:2 * D]
        g_g = gates[:, 2 * D:3 * D]
        o_g = gates[:, 3 * D:4 * D]
        c2 = jax.nn.sigmoid(i_g) * jnp.tanh(g_g)
        if c_st is not None:
            c2 = c2 + jax.nn.sigmoid(f_g) * c_st
        h2 = jax.nn.sigmoid(o_g) * jnp.tanh(c2)
        return h2, c2

    for i in range(NLAY):
        wih = wih_ref[i]
        whh = whh_ref[i]
        bih = bih_ref[i]
        bhh = bhh_ref[i]
        hx = w_ref[i]
        cx_first = None
        for t in range(ST):
            if t == 0:
                hx, cx_first = cell(hx, None, None, wih, whh, bih, bhh)
            else:
                hx, _ = cell(hx, hx, cx_first, wih, whh, bih, bhh)
            hx_ref[t, i] = hx


def _tc_lstm(weights, lstm_W_ih, lstm_W_hh, lstm_b_ih, lstm_b_hh):
    return pl.pallas_call(
        _lstm_body,
        out_shape=jax.ShapeDtypeStruct((ST, NLAY, D, D), jnp.float32),
    )(weights, lstm_W_ih, lstm_W_hh,
      lstm_b_ih.reshape(NLAY, 1, 4 * D), lstm_b_hh.reshape(NLAY, 1, 4 * D))


_RB = 1000


def _adapt_body(x_ref, aw_ref, ab_ref, hx_ref, h_ref, ht_ref):
    h = lax.dot_general(x_ref[...], aw_ref[...], (((1,), (1,)), ((), ())),
                        preferred_element_type=jnp.float32) + ab_ref[...]
    h_ref[...] = h
    ht_ref[...] = jnp.dot(h, hx_ref[...], preferred_element_type=jnp.float32)


def _tc_adapt(x_emb, adapt_W, adapt_b, hx00):
    grid = NN // _RB
    return pl.pallas_call(
        _adapt_body,
        grid=(grid,),
        in_specs=[
            pl.BlockSpec((_RB, D), lambda i: (i, 0)),
            pl.BlockSpec((D, D), lambda i: (0, 0)),
            pl.BlockSpec((1, D), lambda i: (0, 0)),
            pl.BlockSpec((D, D), lambda i: (0, 0)),
        ],
        out_specs=[
            pl.BlockSpec((_RB, D), lambda i: (i, 0)),
            pl.BlockSpec((_RB, D), lambda i: (i, 0)),
        ],
        out_shape=[
            jax.ShapeDtypeStruct((NN, D), jnp.float32),
            jax.ShapeDtypeStruct((NN, D), jnp.float32),
        ],
    )(x_emb, adapt_W, adapt_b.reshape(1, D), hx00)


def _mid_body(agg_ref, hx_ref, ht_ref):
    h1 = jnp.maximum(agg_ref[0] + agg_ref[1], 0.0)
    ht_ref[...] = jnp.dot(h1, hx_ref[...], preferred_element_type=jnp.float32)


def _tc_mid(aggp, hx_next):
    grid = NN // _RB
    return pl.pallas_call(
        _mid_body,
        grid=(grid,),
        in_specs=[
            pl.BlockSpec((2, _RB, D), lambda i: (0, i, 0)),
            pl.BlockSpec((D, D), lambda i: (0, 0)),
        ],
        out_specs=pl.BlockSpec((_RB, D), lambda i: (i, 0)),
        out_shape=jax.ShapeDtypeStruct((NN, D), jnp.float32),
    )(aggp, hx_next)


def _bnd_body(agg_ref, tch_ref, hprev_ref, hx_ref, h_ref, ht_ref):
    h2 = jnp.maximum(agg_ref[0] + agg_ref[1], 0.0)
    tch = jnp.sum(tch_ref[...], axis=0) > 0.0
    h_new = jnp.where(tch, h2, hprev_ref[...])
    h_ref[...] = h_new
    ht_ref[...] = jnp.dot(h_new, hx_ref[...],
                          preferred_element_type=jnp.float32)


def _tc_boundary(aggp, tch, h_prev, hx_next):
    grid = NN // _RB
    return pl.pallas_call(
        _bnd_body,
        grid=(grid,),
        in_specs=[
            pl.BlockSpec((2, _RB, D), lambda i: (0, i, 0)),
            pl.BlockSpec((NW, _RB, 1), lambda i: (0, i, 0)),
            pl.BlockSpec((_RB, D), lambda i: (i, 0)),
            pl.BlockSpec((D, D), lambda i: (0, 0)),
        ],
        out_specs=[
            pl.BlockSpec((_RB, D), lambda i: (i, 0)),
            pl.BlockSpec((_RB, D), lambda i: (i, 0)),
        ],
        out_shape=[
            jax.ShapeDtypeStruct((NN, D), jnp.float32),
            jax.ShapeDtypeStruct((NN, D), jnp.float32),
        ],
    )(aggp, tch, h_prev, hx_next)


def _final_body(agg_ref, tch_ref, hprev_ref, gid_ref, y_ref, ow_ref, ob_ref,
                loss_ref, probs_ref, acc_ref):
    i = pl.program_id(0)

    @pl.when(i == 0)
    def _():
        acc_ref[...] = jnp.full((NG, D), -jnp.inf, jnp.float32)

    h2 = jnp.maximum(agg_ref[0] + agg_ref[1], 0.0)
    tch = jnp.sum(tch_ref[...], axis=0) > 0.0
    h_new = jnp.where(tch, h2, hprev_ref[...])
    gid = gid_ref[...]
    for g in range(NG):
        m = jnp.where(gid == g, h_new, -jnp.inf)
        acc_ref[g:g + 1, :] = jnp.maximum(
            acc_ref[g:g + 1, :], jnp.max(m, axis=0, keepdims=True))

    @pl.when(i == pl.num_programs(0) - 1)
    def _():
        pooled = acc_ref[...]
        logits = (jnp.sum(pooled * ow_ref[...], axis=1, keepdims=True)
                  + ob_ref[0, 0])
        x = logits
        y = y_ref[...]
        per = (jnp.maximum(x, 0.0) - x * y
               + jnp.log(1.0 + jnp.exp(-jnp.abs(x))))
        loss_ref[...] = jnp.sum(per, axis=0, keepdims=True) / NG
        probs_ref[...] = jax.nn.sigmoid(logits)


def _tc_final(aggp, tch, h_prev, gid, y, out_W, out_b):
    grid = NN // _RB
    return pl.pallas_call(
        _final_body,
        grid=(grid,),
        in_specs=[
            pl.BlockSpec((2, _RB, D), lambda i: (0, i, 0)),
            pl.BlockSpec((NW, _RB, 1), lambda i: (0, i, 0)),
            pl.BlockSpec((_RB, D), lambda i: (i, 0)),
            pl.BlockSpec((_RB, 1), lambda i: (i, 0)),
            pl.BlockSpec((NG, 1), lambda i: (0, 0)),
            pl.BlockSpec((1, D), lambda i: (0, 0)),
            pl.BlockSpec((1, 1), lambda i: (0, 0)),
        ],
        out_specs=[
            pl.BlockSpec((1, 1), lambda i: (0, 0)),
            pl.BlockSpec((NG, 1), lambda i: (0, 0)),
        ],
        out_shape=[
            jax.ShapeDtypeStruct((1, 1), jnp.float32),
            jax.ShapeDtypeStruct((NG, 1), jnp.float32),
        ],
        scratch_shapes=[pltpu.VMEM((NG, D), jnp.float32)],
    )(aggp, tch, h_prev, gid, y, out_W, out_b)


def kernel(node_ids, edge_src, edge_dst, edge_time, edge_weight, graph_ids,
           y_data, word_embeds, adapt_W, adapt_b, weights, lstm_W_ih,
           lstm_W_hh, lstm_b_ih, lstm_b_hh, out_W, out_b):
    node_ids = node_ids.astype(jnp.int32)
    edge_src = edge_src.astype(jnp.int32)
    edge_dst = edge_dst.astype(jnp.int32)
    edge_time = edge_time.astype(jnp.int32)
    gid = graph_ids.astype(jnp.int32).reshape(NN, 1)

    x_emb, touched_p, psrc, pdst, pwgt, offcnt = _sc_embed_touched(
        word_embeds, node_ids, edge_src, edge_dst, edge_time, edge_weight)

    hx_all = _tc_lstm(weights, lstm_W_ih, lstm_W_hh, lstm_b_ih, lstm_b_hh)
    hx = [[hx_all[t, i] for i in range(NLAY)] for t in range(ST)]

    h, ht = _tc_adapt(x_emb, adapt_W, adapt_b, hx[0][0])

    zeros = jnp.zeros((NN, D), jnp.float32)
    touched_r = touched_p.reshape(NW, ST, NN, 1)

    loss = probs = None
    for t in range(ST):
        for i in range(NLAY):
            aggp = _sc_scatter_t[t](ht, psrc, pdst, pwgt, offcnt, zeros)
            if i + 1 < NLAY:
                ht = _tc_mid(aggp, hx[t][i + 1])
            elif t + 1 < ST:
                h, ht = _tc_boundary(aggp, touched_r[:, t], h, hx[t + 1][0])
            else:
                loss, probs = _tc_final(
                    aggp, touched_r[:, t], h, gid,
                    y_data.reshape(NG, 1), out_W, out_b.reshape(1, 1))

    return loss.reshape(()), probs


# speedup vs baseline: 1.6914x; 1.6683x over previous
import functools

import jax
import jax.numpy as jnp
from jax import lax
from jax.experimental import pallas as pl
from jax.experimental.pallas import tpu as pltpu
from jax.experimental.pallas import tpu_sc as plsc

NN = 10000
NE = 320000
D = 128
ST = 4
NLAY = 2
NG = 16

NW = 32
EPW = NE // NW
CH = 80
NCH = EPW // CH
RPW = 624
CHS = 64
RCAP = 10240

_mesh = plsc.VectorSubcoreMesh(core_axis_name="c", subcore_axis_name="s")


@functools.partial(
    pl.kernel,
    mesh=_mesh,
    out_type=[
        jax.ShapeDtypeStruct((NN, D), jnp.float32),
        jax.ShapeDtypeStruct((NW * ST * NN,), jnp.float32),
        jax.ShapeDtypeStruct((NW * RCAP,), jnp.int32),
        jax.ShapeDtypeStruct((NW * RCAP,), jnp.int32),
        jax.ShapeDtypeStruct((NW * RCAP,), jnp.float32),
        jax.ShapeDtypeStruct((NW * 16,), jnp.int32),
    ],
    scratch_types=[
        pltpu.VMEM((CH,), jnp.int32),
        pltpu.VMEM((CH, D), jnp.float32),
        pltpu.VMEM((EPW,), jnp.int32),
        pltpu.VMEM((EPW,), jnp.int32),
        pltpu.VMEM((EPW,), jnp.int32),
        pltpu.VMEM((EPW,), jnp.float32),
        pltpu.VMEM((RCAP,), jnp.int32),
        pltpu.VMEM((RCAP,), jnp.int32),
        pltpu.VMEM((RCAP,), jnp.float32),
        pltpu.VMEM((ST * NN,), jnp.float32),
        pltpu.VMEM((16,), jnp.int32),
        pltpu.SemaphoreType.DMA,
    ],
    compiler_params=pltpu.CompilerParams(needs_layout_passes=False),
)
def _sc_embed_touched(emb_hbm, ids_hbm, src_hbm, dst_hbm, time_hbm, w_hbm,
                      x_out, touched_out, ps_out, pd_out, pw_out, oc_out,
                      idx_v, rows_v, es_v, ed_v, et_v, ew_v,
                      ps_v, pd_v, pw_v, tfl_v, oc_v, sem):
    c = lax.axis_index("c")
    s = lax.axis_index("s")
    wid = c * 16 + s

    nchunk_all = NN // CH
    z16 = jnp.zeros((16,), jnp.float32)

    def emb_body(i, _):
        c2 = wid + i * NW
        base = c2 * CH
        pltpu.sync_copy(ids_hbm.at[pl.ds(base, CH)], idx_v)
        pltpu.async_copy(emb_hbm.at[idx_v], rows_v, sem).wait()
        pltpu.sync_copy(rows_v, x_out.at[pl.ds(base, CH)])
        return 0

    my_chunks = (nchunk_all - wid + NW - 1) // NW
    lax.fori_loop(0, my_chunks, emb_body, 0)

    ebase = wid * EPW
    pltpu.sync_copy(src_hbm.at[pl.ds(ebase, EPW)], es_v)
    pltpu.sync_copy(dst_hbm.at[pl.ds(ebase, EPW)], ed_v)
    pltpu.sync_copy(time_hbm.at[pl.ds(ebase, EPW)], et_v)
    pltpu.sync_copy(w_hbm.at[pl.ds(ebase, EPW)], ew_v)

    def zero_body(k, _):
        tfl_v[pl.ds(k * 16, 16)] = z16
        return 0
    lax.fori_loop(0, ST * NN // 16, zero_body, 0)

    ones = jnp.ones((16,), jnp.float32)

    def edge_body(v, _):
        sl = pl.ds(v * 16, 16)
        tbase = et_v[sl] * NN
        plsc.addupdate_scatter(tfl_v, [tbase + es_v[sl]], ones)
        plsc.addupdate_scatter(tfl_v, [tbase + ed_v[sl]], ones)
        return 0

    lax.fori_loop(0, EPW // 16, edge_body, 0)
    pltpu.sync_copy(tfl_v, touched_out.at[pl.ds(wid * (ST * NN), ST * NN)])

    iota16 = jnp.arange(16, dtype=jnp.int32)

    def pre_body(k, _):
        sl = pl.ds(k * 16, 16)
        spread = (iota16 * 599 + k * 23 + wid * 311) % jnp.int32(NN)
        ps_v[sl] = spread
        pd_v[sl] = spread
        pw_v[sl] = z16
        return 0
    lax.fori_loop(0, RCAP // 16, pre_body, 0)
    offcnt = []
    pos = jnp.int32(0)
    for t in range(ST):
        pos = ((pos + CHS - 1) // CHS) * CHS
        seg_start = pos

        def comp_body(v, p, t=t):
            sl = pl.ds(v * 16, 16)
            tv = et_v[sl]
            m = tv == t
            csum = jnp.cumsum(m.astype(jnp.int32))
            dpos = p + csum - 1
            plsc.store_scatter(ps_v, [dpos], es_v[sl], mask=m)
            plsc.store_scatter(pd_v, [dpos], ed_v[sl], mask=m)
            plsc.store_scatter(pw_v, [dpos], ew_v[sl], mask=m)
            return p + csum[15]

        pos = lax.fori_loop(0, EPW // 16, comp_body, pos)
        offcnt.append(seg_start)
        offcnt.append(pos - seg_start)

    pltpu.sync_copy(ps_v, ps_out.at[pl.ds(wid * RCAP, RCAP)])
    pltpu.sync_copy(pd_v, pd_out.at[pl.ds(wid * RCAP, RCAP)])
    pltpu.sync_copy(pw_v, pw_out.at[pl.ds(wid * RCAP, RCAP)])
    ocv = jnp.zeros((16,), jnp.int32)
    for k, val in enumerate(offcnt):
        ocv = jnp.where(iota16 == k, val, ocv)
    oc_v[...] = ocv
    pltpu.sync_copy(oc_v, oc_out.at[pl.ds(wid * 16, 16)])


def _make_sc_scatter(t):
    @functools.partial(
        pl.kernel,
        mesh=_mesh,
        out_type=jax.ShapeDtypeStruct((2, NN, D), jnp.float32),
        scratch_types=[
            pltpu.VMEM((RCAP,), jnp.int32),
            pltpu.VMEM((RCAP,), jnp.int32),
            pltpu.VMEM((RCAP,), jnp.float32),
            pltpu.VMEM((CHS,), jnp.int32),
            pltpu.VMEM((CHS,), jnp.int32),
            pltpu.VMEM((CHS, D), jnp.float32),
            pltpu.VMEM((CHS,), jnp.int32),
            pltpu.VMEM((CHS,), jnp.int32),
            pltpu.VMEM((CHS, D), jnp.float32),
            pltpu.VMEM((16,), jnp.int32),
            pltpu.VMEM_SHARED((NN, D), jnp.float32),
            pltpu.SemaphoreType.DMA,
            pltpu.SemaphoreType.DMA,
            pltpu.SemaphoreType.DMA,
            pltpu.SemaphoreType.DMA,
        ],
    )
    def _sc_scatter(ht_hbm, ps_hbm, pd_hbm, pw_hbm, oc_hbm, zeros_hbm,
                    agg_out, rs_v, rd_v, rw_v,
                    src_a, dst_a, rows_a, src_b, dst_b, rows_b,
                    oc_v, agg_sh, sem_a, sem_b, sem_sa, sem_sb):
        c = lax.axis_index("c")
        s = lax.axis_index("s")
        wid = c * 16 + s

        pltpu.sync_copy(zeros_hbm.at[pl.ds(s * RPW, RPW)],
                        agg_sh.at[pl.ds(s * RPW, RPW)])

        @pl.when(s == 15)
        def _():
            pltpu.sync_copy(zeros_hbm.at[pl.ds(16 * RPW, NN - 16 * RPW)],
                            agg_sh.at[pl.ds(16 * RPW, NN - 16 * RPW)])

        pltpu.sync_copy(oc_hbm.at[pl.ds(wid * 16, 16)], oc_v)
        pltpu.async_copy(ps_hbm.at[pl.ds(wid * RCAP, RCAP)], rs_v, sem_a)
        pltpu.async_copy(pd_hbm.at[pl.ds(wid * RCAP, RCAP)], rd_v, sem_a)
        pltpu.async_copy(pw_hbm.at[pl.ds(wid * RCAP, RCAP)], rw_v, sem_a)
        pltpu.make_async_copy(ps_hbm.at[pl.ds(wid * RCAP, RCAP)], rs_v,
                              sem_a).wait()
        pltpu.make_async_copy(pd_hbm.at[pl.ds(wid * RCAP, RCAP)], rd_v,
                              sem_a).wait()
        pltpu.make_async_copy(pw_hbm.at[pl.ds(wid * RCAP, RCAP)], rw_v,
                              sem_a).wait()
        ocv = oc_v[...]
        off = pl.multiple_of(ocv[2 * t], CHS)
        cnt = ocv[2 * t + 1]
        plsc.subcore_barrier()

        nchunks = (cnt + CHS - 1) // CHS

        def fire(src_v, rows_v, sem):
            pltpu.async_copy(ht_hbm.at[src_v], rows_v, sem)

        def prep(c2, src_v, dst_v):
            cbase = off + c2 * CHS
            for v in range(CHS // 16):
                osl = pl.ds(v * 16, 16)
                src_v[osl] = rs_v[pl.ds(cbase + v * 16, 16)]
                dst_v[osl] = rd_v[pl.ds(cbase + v * 16, 16)]

        def drain(src_v, rows_v, sem):
            pltpu.make_async_copy(ht_hbm.at[src_v], rows_v, sem).wait()

        def scale(c2, rows_v):
            cbase = off + c2 * CHS

            def scale_body(g, _):
                wv = rw_v[pl.ds(cbase + g * 16, 16)]
                for l in range(16):
                    e = g * 16 + l
                    we = wv[l]
                    for j in range(D // 16):
                        sl = pl.ds(j * 16, 16)
                        rows_v[e, sl] = rows_v[e, sl] * we
                return 0
            lax.fori_loop(0, CHS // 16, scale_body, 0)

        def consume(c2, dst_v, rows_v):
            scale(c2, rows_v)
            pltpu.sync_copy(rows_v, agg_sh.at[dst_v], add=True)

        @pl.when(nchunks > 0)
        def _():
            prep(jnp.int32(0), src_a, dst_a)
            fire(src_a, rows_a, sem_a)

        def pair_body(i, _):
            a = 2 * i
            b = a + 1

            @pl.when(b < nchunks)
            def _():
                prep(b, src_b, dst_b)
                fire(src_b, rows_b, sem_b)
            drain(src_a, rows_a, sem_a)
            consume(a, dst_a, rows_a)

            @pl.when(a + 2 < nchunks)
            def _():
                prep(a + 2, src_a, dst_a)
                fire(src_a, rows_a, sem_a)

            @pl.when(b < nchunks)
            def _():
                drain(src_b, rows_b, sem_b)
                consume(b, dst_b, rows_b)
            return 0

        lax.fori_loop(0, (nchunks + 1) // 2, pair_body, 0)
        plsc.subcore_barrier()

        pltpu.sync_copy(agg_sh.at[pl.ds(s * RPW, RPW)],
                        agg_out.at[c, pl.ds(s * RPW, RPW)])

        @pl.when(s == 15)
        def _():
            pltpu.sync_copy(agg_sh.at[pl.ds(16 * RPW, NN - 16 * RPW)],
                            agg_out.at[c, pl.ds(16 * RPW, NN - 16 * RPW)])

    return _sc_scatter


_sc_scatter_t = [_make_sc_scatter(t) for t in range(ST)]


def _lstm_body(w_ref, wih_ref, whh_ref, bih_ref, bhh_ref, hx_ref):
    dn_t = (((1,), (1,)), ((), ()))

    def cell(x, h, c_st, wih, whh, bih, bhh):
        gates = lax.dot_general(x, wih, dn_t,
                                preferred_element_type=jnp.float32) + bih
        if h is not None:
            gates = gates + lax.dot_general(h, whh, dn_t,
                                            preferred_element_type=jnp.float32)
        gates = gates + bhh
        i_g = gates[:, 0 * D:1 * D]
        f_g = gates[:, 1 * D:2 * D]
        g_g = gates[:, 2 * D:3 * D]
        o_g = gates[:, 3 * D:4 * D]
        c2 = jax.nn.sigmoid(i_g) * jnp.tanh(g_g)
        if c_st is not None:
            c2 = c2 + jax.nn.sigmoid(f_g) * c_st
        h2 = jax.nn.sigmoid(o_g) * jnp.tanh(c2)
        return h2, c2

    for i in range(NLAY):
        wih = wih_ref[i]
        whh = whh_ref[i]
        bih = bih_ref[i]
        bhh = bhh_ref[i]
        hx = w_ref[i]
        cx_first = None
        for t in range(ST):
            if t == 0:
                hx, cx_first = cell(hx, None, None, wih, whh, bih, bhh)
            else:
                hx, _ = cell(hx, hx, cx_first, wih, whh, bih, bhh)
            hx_ref[t, i] = hx


def _tc_lstm(weights, lstm_W_ih, lstm_W_hh, lstm_b_ih, lstm_b_hh):
    return pl.pallas_call(
        _lstm_body,
        out_shape=jax.ShapeDtypeStruct((ST, NLAY, D, D), jnp.float32),
    )(weights, lstm_W_ih, lstm_W_hh,
      lstm_b_ih.reshape(NLAY, 1, 4 * D), lstm_b_hh.reshape(NLAY, 1, 4 * D))


_RB = 1000


def _adapt_body(x_ref, aw_ref, ab_ref, hx_ref, h_ref, ht_ref):
    h = lax.dot_general(x_ref[...], aw_ref[...], (((1,), (1,)), ((), ())),
                        preferred_element_type=jnp.float32) + ab_ref[...]
    h_ref[...] = h
    ht_ref[...] = jnp.dot(h, hx_ref[...], preferred_element_type=jnp.float32)


def _tc_adapt(x_emb, adapt_W, adapt_b, hx00):
    grid = NN // _RB
    return pl.pallas_call(
        _adapt_body,
        grid=(grid,),
        in_specs=[
            pl.BlockSpec((_RB, D), lambda i: (i, 0)),
            pl.BlockSpec((D, D), lambda i: (0, 0)),
            pl.BlockSpec((1, D), lambda i: (0, 0)),
            pl.BlockSpec((D, D), lambda i: (0, 0)),
        ],
        out_specs=[
            pl.BlockSpec((_RB, D), lambda i: (i, 0)),
            pl.BlockSpec((_RB, D), lambda i: (i, 0)),
        ],
        out_shape=[
            jax.ShapeDtypeStruct((NN, D), jnp.float32),
            jax.ShapeDtypeStruct((NN, D), jnp.float32),
        ],
    )(x_emb, adapt_W, adapt_b.reshape(1, D), hx00)


def _tred_body(tp_ref, out_ref):
    out_ref[...] = jnp.sum(tp_ref[...], axis=0, keepdims=True)


def _tc_touchred(touched_p):
    return pl.pallas_call(
        _tred_body,
        out_shape=jax.ShapeDtypeStruct((1, ST * NN), jnp.float32),
    )(touched_p)


def _mid_body(agg_ref, hx_ref, ht_ref):
    h1 = jnp.maximum(agg_ref[0] + agg_ref[1], 0.0)
    ht_ref[...] = jnp.dot(h1, hx_ref[...], preferred_element_type=jnp.float32)


def _tc_mid(aggp, hx_next):
    grid = NN // _RB
    return pl.pallas_call(
        _mid_body,
        grid=(grid,),
        in_specs=[
            pl.BlockSpec((2, _RB, D), lambda i: (0, i, 0)),
            pl.BlockSpec((D, D), lambda i: (0, 0)),
        ],
        out_specs=pl.BlockSpec((_RB, D), lambda i: (i, 0)),
        out_shape=jax.ShapeDtypeStruct((NN, D), jnp.float32),
    )(aggp, hx_next)


def _bnd_body(agg_ref, tch_ref, hprev_ref, hx_ref, h_ref, ht_ref):
    h2 = jnp.maximum(agg_ref[0] + agg_ref[1], 0.0)
    tch = tch_ref[...] > 0.0
    h_new = jnp.where(tch, h2, hprev_ref[...])
    h_ref[...] = h_new
    ht_ref[...] = jnp.dot(h_new, hx_ref[...],
                          preferred_element_type=jnp.float32)


def _tc_boundary(aggp, tch, h_prev, hx_next):
    grid = NN // _RB
    return pl.pallas_call(
        _bnd_body,
        grid=(grid,),
        in_specs=[
            pl.BlockSpec((2, _RB, D), lambda i: (0, i, 0)),
            pl.BlockSpec((_RB, 1), lambda i: (i, 0)),
            pl.BlockSpec((_RB, D), lambda i: (i, 0)),
            pl.BlockSpec((D, D), lambda i: (0, 0)),
        ],
        out_specs=[
            pl.BlockSpec((_RB, D), lambda i: (i, 0)),
            pl.BlockSpec((_RB, D), lambda i: (i, 0)),
        ],
        out_shape=[
            jax.ShapeDtypeStruct((NN, D), jnp.float32),
            jax.ShapeDtypeStruct((NN, D), jnp.float32),
        ],
    )(aggp, tch, h_prev, hx_next)


def _final_body(agg_ref, tch_ref, hprev_ref, gid_ref, y_ref, ow_ref, ob_ref,
                loss_ref, probs_ref, acc_ref):
    i = pl.program_id(0)

    @pl.when(i == 0)
    def _():
        acc_ref[...] = jnp.full((NG, D), -jnp.inf, jnp.float32)

    h2 = jnp.maximum(agg_ref[0] + agg_ref[1], 0.0)
    tch = tch_ref[...] > 0.0
    h_new = jnp.where(tch, h2, hprev_ref[...])
    gid = gid_ref[...]
    for g in range(NG):
        m = jnp.where(gid == g, h_new, -jnp.inf)
        acc_ref[g:g + 1, :] = jnp.maximum(
            acc_ref[g:g + 1, :], jnp.max(m, axis=0, keepdims=True))

    @pl.when(i == pl.num_programs(0) - 1)
    def _():
        pooled = acc_ref[...]
        logits = (jnp.sum(pooled * ow_ref[...], axis=1, keepdims=True)
                  + ob_ref[0, 0])
        x = logits
        y = y_ref[...]
        per = (jnp.maximum(x, 0.0) - x * y
               + jnp.log(1.0 + jnp.exp(-jnp.abs(x))))
        loss_ref[...] = jnp.sum(per, axis=0, keepdims=True) / NG
        probs_ref[...] = jax.nn.sigmoid(logits)


def _tc_final(aggp, tch, h_prev, gid, y, out_W, out_b):
    grid = NN // _RB
    return pl.pallas_call(
        _final_body,
        grid=(grid,),
        in_specs=[
            pl.BlockSpec((2, _RB, D), lambda i: (0, i, 0)),
            pl.BlockSpec((_RB, 1), lambda i: (i, 0)),
            pl.BlockSpec((_RB, D), lambda i: (i, 0)),
            pl.BlockSpec((_RB, 1), lambda i: (i, 0)),
            pl.BlockSpec((NG, 1), lambda i: (0, 0)),
            pl.BlockSpec((1, D), lambda i: (0, 0)),
            pl.BlockSpec((1, 1), lambda i: (0, 0)),
        ],
        out_specs=[
            pl.BlockSpec((1, 1), lambda i: (0, 0)),
            pl.BlockSpec((NG, 1), lambda i: (0, 0)),
        ],
        out_shape=[
            jax.ShapeDtypeStruct((1, 1), jnp.float32),
            jax.ShapeDtypeStruct((NG, 1), jnp.float32),
        ],
        scratch_shapes=[pltpu.VMEM((NG, D), jnp.float32)],
    )(aggp, tch, h_prev, gid, y, out_W, out_b)


def kernel(node_ids, edge_src, edge_dst, edge_time, edge_weight, graph_ids,
           y_data, word_embeds, adapt_W, adapt_b, weights, lstm_W_ih,
           lstm_W_hh, lstm_b_ih, lstm_b_hh, out_W, out_b):
    node_ids = node_ids.astype(jnp.int32)
    edge_src = edge_src.astype(jnp.int32)
    edge_dst = edge_dst.astype(jnp.int32)
    edge_time = edge_time.astype(jnp.int32)
    gid = graph_ids.astype(jnp.int32).reshape(NN, 1)

    x_emb, touched_p, psrc, pdst, pwgt, offcnt = _sc_embed_touched(
        word_embeds, node_ids, edge_src, edge_dst, edge_time, edge_weight)

    hx_all = _tc_lstm(weights, lstm_W_ih, lstm_W_hh, lstm_b_ih, lstm_b_hh)
    hx = [[hx_all[t, i] for i in range(NLAY)] for t in range(ST)]

    h, ht = _tc_adapt(x_emb, adapt_W, adapt_b, hx[0][0])

    zeros = jnp.zeros((NN, D), jnp.float32)
    touched_sum = _tc_touchred(touched_p.reshape(NW, ST * NN))
    touched_r = touched_sum.reshape(ST, NN, 1)

    loss = probs = None
    for t in range(ST):
        for i in range(NLAY):
            aggp = _sc_scatter_t[t](ht, psrc, pdst, pwgt, offcnt, zeros)
            if i + 1 < NLAY:
                ht = _tc_mid(aggp, hx[t][i + 1])
            elif t + 1 < ST:
                h, ht = _tc_boundary(aggp, touched_r[t], h, hx[t + 1][0])
            else:
                loss, probs = _tc_final(
                    aggp, touched_r[t], h, gid,
                    y_data.reshape(NG, 1), out_W, out_b.reshape(1, 1))

    return loss.reshape(()), probs
